# baseline (speedup 1.0000x reference)
import jax
import jax.numpy as jnp
from jax.experimental import pallas as pl
from jax.experimental.pallas import tpu as pltpu

EMBED_DIM = 4096
NUM_EXPERTS = 64
TOP_K = 8
BT = 512


def _router_block(x_ref, w_ref, b_ref, wout_ref, iout_ref, pout_ref):
    xb = x_ref[...]
    w = w_ref[...]
    lt = jax.lax.dot_general(
        w, xb, (((1,), (1,)), ((), ())),
        preferred_element_type=jnp.float32,
    ) + b_ref[...]

    m = jnp.max(lt, axis=0, keepdims=True)
    e = jnp.exp(lt - m)
    s = jnp.sum(e, axis=0, keepdims=True)
    pt = e / s
    pout_ref[...] = pt.T

    sub = jax.lax.broadcasted_iota(jnp.int32, pt.shape, 0)
    p = pt
    w_rows = []
    i_rows = []
    for _ in range(TOP_K):
        mx = jnp.max(p, axis=0, keepdims=True)
        idx = jnp.min(jnp.where(p == mx, sub, NUM_EXPERTS),
                      axis=0, keepdims=True)
        w_rows.append(mx)
        i_rows.append(idx)
        p = jnp.where(sub == idx, -1.0, p)
    tw = jnp.concatenate(w_rows, axis=0)
    ti = jnp.concatenate(i_rows, axis=0)
    tw = tw / (jnp.sum(tw, axis=0, keepdims=True) + 1e-09)
    wout_ref[...] = tw.T
    iout_ref[...] = ti.T


@jax.jit
def kernel(x, W, b):
    x = x.reshape(-1, EMBED_DIM)
    n = x.shape[0]
    grid = (n // BT,)
    b_col = b.reshape(NUM_EXPERTS, 1)
    out_shapes = (
        jax.ShapeDtypeStruct((n, TOP_K), jnp.float32),
        jax.ShapeDtypeStruct((n, TOP_K), jnp.int32),
        jax.ShapeDtypeStruct((n, NUM_EXPERTS), jnp.float32),
    )
    tw, ti, probs = pl.pallas_call(
        _router_block,
        grid=grid,
        in_specs=[
            pl.BlockSpec((BT, EMBED_DIM), lambda i: (i, 0)),
            pl.BlockSpec((NUM_EXPERTS, EMBED_DIM), lambda i: (0, 0)),
            pl.BlockSpec((NUM_EXPERTS, 1), lambda i: (0, 0)),
        ],
        out_specs=(
            pl.BlockSpec((BT, TOP_K), lambda i: (i, 0)),
            pl.BlockSpec((BT, TOP_K), lambda i: (i, 0)),
            pl.BlockSpec((BT, NUM_EXPERTS), lambda i: (i, 0)),
        ),
        out_shape=out_shapes,
        compiler_params=pltpu.CompilerParams(
            dimension_semantics=("parallel",),
        ),
    )(x, W, b_col)
    return (tw, ti, probs)


# speedup vs baseline: 1.0763x; 1.0763x over previous
import jax
import jax.numpy as jnp
from jax.experimental import pallas as pl
from jax.experimental.pallas import tpu as pltpu

EMBED_DIM = 4096
NUM_EXPERTS = 64
TOP_K = 8
BT = 1024


def _router_block(x_ref, w_ref, b_ref, wout_ref, iout_ref, pout_ref):
    xb = x_ref[...]
    w = w_ref[...]
    lt = jax.lax.dot_general(
        w, xb, (((1,), (1,)), ((), ())),
        preferred_element_type=jnp.float32,
    ) + b_ref[...]

    m = jnp.max(lt, axis=0, keepdims=True)
    e = jnp.exp(lt - m)
    s = jnp.sum(e, axis=0, keepdims=True)
    pt = e / s
    pout_ref[...] = pt.T

    sub = jax.lax.broadcasted_iota(jnp.int32, pt.shape, 0)
    p = pt
    w_rows = []
    i_rows = []
    for _ in range(TOP_K):
        mx = jnp.max(p, axis=0, keepdims=True)
        idx = jnp.min(jnp.where(p == mx, sub, NUM_EXPERTS),
                      axis=0, keepdims=True)
        w_rows.append(mx)
        i_rows.append(idx)
        p = jnp.where(sub == idx, -1.0, p)
    tw = jnp.concatenate(w_rows, axis=0)
    ti = jnp.concatenate(i_rows, axis=0)
    tw = tw / (jnp.sum(tw, axis=0, keepdims=True) + 1e-09)
    wout_ref[...] = tw.T
    iout_ref[...] = ti.T


@jax.jit
def kernel(x, W, b):
    x = x.reshape(-1, EMBED_DIM)
    n = x.shape[0]
    grid = (n // BT,)
    b_col = b.reshape(NUM_EXPERTS, 1)
    out_shapes = (
        jax.ShapeDtypeStruct((n, TOP_K), jnp.float32),
        jax.ShapeDtypeStruct((n, TOP_K), jnp.int32),
        jax.ShapeDtypeStruct((n, NUM_EXPERTS), jnp.float32),
    )
    tw, ti, probs = pl.pallas_call(
        _router_block,
        grid=grid,
        in_specs=[
            pl.BlockSpec((BT, EMBED_DIM), lambda i: (i, 0)),
            pl.BlockSpec((NUM_EXPERTS, EMBED_DIM), lambda i: (0, 0)),
            pl.BlockSpec((NUM_EXPERTS, 1), lambda i: (0, 0)),
        ],
        out_specs=(
            pl.BlockSpec((BT, TOP_K), lambda i: (i, 0)),
            pl.BlockSpec((BT, TOP_K), lambda i: (i, 0)),
            pl.BlockSpec((BT, NUM_EXPERTS), lambda i: (i, 0)),
        ),
        out_shape=out_shapes,
        compiler_params=pltpu.CompilerParams(
            dimension_semantics=("parallel",),
        ),
    )(x, W, b_col)
    return (tw, ti, probs)
